# baseline (speedup 1.0000x reference)
import functools

import jax
import jax.numpy as jnp
from jax import lax
from jax.experimental import pallas as pl
from jax.experimental.pallas import tpu as pltpu
from jax.experimental.pallas import tpu_sc as plsc

N = 10000
E = 320000
D = 128
NG = 64
NC = 2
NS = 16
NW = NC * NS
CH = 128
K = -(-E // (NW * CH))
K0 = 53
K1 = 2 * K - K0
EPAD = NS * CH * (K0 + K1)
RPT = 632
NACC = NS * RPT
NT = N + 8


def _sc_mesh():
    return plsc.VectorSubcoreMesh(
        core_axis_name="c", subcore_axis_name="s", num_cores=NC, num_subcores=NS
    )


def _scatter_body(g_hbm, sdl, zrows, out_hbm, sd0, sd1, rows0, rows1, acc,
                  isem0, isem1, gsem0, gsem1):
    c = lax.axis_index("c")
    s = lax.axis_index("s")
    kc = jnp.where(c == 0, K0, K1)
    pltpu.sync_copy(zrows, acc.at[pl.ds(s * RPT, RPT)])
    plsc.subcore_barrier()

    pltpu.sync_copy(sdl.at[c, s, 0], sd0)
    pltpu.async_copy(g_hbm.at[sd0.at[0]], rows0, gsem0)
    pltpu.async_copy(sdl.at[c, s, 1], sd1, isem1)

    def pair(i, carry):
        k0 = i * 2
        pltpu.make_async_copy(sdl.at[c, s, k0 + 1], sd1, isem1).wait()
        pltpu.async_copy(g_hbm.at[sd1.at[0]], rows1, gsem1)
        pltpu.make_async_copy(g_hbm.at[sd0.at[0]], rows0, gsem0).wait()
        pltpu.sync_copy(rows0, acc.at[sd0.at[1]], add=True)
        pltpu.async_copy(sdl.at[c, s, k0 + 2], sd0, isem0)
        pltpu.make_async_copy(sdl.at[c, s, k0 + 2], sd0, isem0).wait()
        pltpu.async_copy(g_hbm.at[sd0.at[0]], rows0, gsem0)
        pltpu.make_async_copy(g_hbm.at[sd1.at[0]], rows1, gsem1).wait()
        pltpu.sync_copy(rows1, acc.at[sd1.at[1]], add=True)
        pltpu.async_copy(sdl.at[c, s, k0 + 3], sd1, isem1)
        return carry

    lax.fori_loop(0, (kc - 3) // 2, pair, 0)
    pltpu.make_async_copy(sdl.at[c, s, kc - 2], sd1, isem1).wait()
    pltpu.async_copy(g_hbm.at[sd1.at[0]], rows1, gsem1)
    pltpu.make_async_copy(g_hbm.at[sd0.at[0]], rows0, gsem0).wait()
    pltpu.sync_copy(rows0, acc.at[sd0.at[1]], add=True)
    pltpu.sync_copy(sdl.at[c, s, kc - 1], sd0)
    pltpu.async_copy(g_hbm.at[sd0.at[0]], rows0, gsem0)
    pltpu.make_async_copy(g_hbm.at[sd1.at[0]], rows1, gsem1).wait()
    pltpu.sync_copy(rows1, acc.at[sd1.at[1]], add=True)
    pltpu.make_async_copy(g_hbm.at[sd0.at[0]], rows0, gsem0).wait()
    pltpu.sync_copy(rows0, acc.at[sd0.at[1]], add=True)
    plsc.subcore_barrier()
    pltpu.sync_copy(acc.at[pl.ds(s * RPT, RPT)], out_hbm.at[c, pl.ds(s * RPT, RPT)])


def _edge_scatter(g, sdl, zrows):
    return pl.kernel(
        _scatter_body,
        out_type=jax.ShapeDtypeStruct((NC, NACC, D), jnp.float32),
        mesh=_sc_mesh(),
        scratch_types=[
            pltpu.VMEM((2, CH), jnp.int32),
            pltpu.VMEM((2, CH), jnp.int32),
            pltpu.VMEM((CH, D), jnp.float32),
            pltpu.VMEM((CH, D), jnp.float32),
            pltpu.VMEM_SHARED((NACC, D), jnp.float32),
            pltpu.SemaphoreType.DMA,
            pltpu.SemaphoreType.DMA,
            pltpu.SemaphoreType.DMA,
            pltpu.SemaphoreType.DMA,
        ],
    )(g, sdl, zrows)


def _deg_body(ones_hbm, sdl, zrows, out_hbm, sd0, sd1, rows0, acc, isem0, isem1):
    c = lax.axis_index("c")
    s = lax.axis_index("s")
    kc = jnp.where(c == 0, K0, K1)
    pltpu.sync_copy(zrows, acc.at[pl.ds(s * RPT, RPT)])
    pltpu.sync_copy(ones_hbm.at[pl.ds(0, CH)], rows0)
    plsc.subcore_barrier()

    pltpu.sync_copy(sdl.at[c, s, 0], sd0)
    pltpu.async_copy(sdl.at[c, s, 1], sd1, isem1)

    def pair(i, carry):
        k0 = i * 2
        pltpu.sync_copy(rows0, acc.at[sd0.at[1]], add=True)
        pltpu.async_copy(sdl.at[c, s, k0 + 2], sd0, isem0)
        pltpu.make_async_copy(sdl.at[c, s, k0 + 1], sd1, isem1).wait()
        pltpu.sync_copy(rows0, acc.at[sd1.at[1]], add=True)
        pltpu.async_copy(sdl.at[c, s, k0 + 3], sd1, isem1)
        pltpu.make_async_copy(sdl.at[c, s, k0 + 2], sd0, isem0).wait()
        return carry

    lax.fori_loop(0, (kc - 3) // 2, pair, 0)
    pltpu.sync_copy(rows0, acc.at[sd0.at[1]], add=True)
    pltpu.make_async_copy(sdl.at[c, s, kc - 2], sd1, isem1).wait()
    pltpu.sync_copy(rows0, acc.at[sd1.at[1]], add=True)
    pltpu.sync_copy(sdl.at[c, s, kc - 1], sd0)
    pltpu.sync_copy(rows0, acc.at[sd0.at[1]], add=True)
    plsc.subcore_barrier()
    pltpu.sync_copy(acc.at[pl.ds(s * RPT, RPT)], out_hbm.at[c, pl.ds(s * RPT, RPT)])


def _edge_degree(ones, sdl, zrows):
    return pl.kernel(
        _deg_body,
        out_type=jax.ShapeDtypeStruct((NC, NACC, D), jnp.float32),
        mesh=_sc_mesh(),
        scratch_types=[
            pltpu.VMEM((2, CH), jnp.int32),
            pltpu.VMEM((2, CH), jnp.int32),
            pltpu.VMEM((CH, D), jnp.float32),
            pltpu.VMEM_SHARED((NACC, D), jnp.float32),
            pltpu.SemaphoreType.DMA,
            pltpu.SemaphoreType.DMA,
        ],
    )(ones, sdl, zrows)


PAD = EPAD - E


def _dinv_from(pdeg_ref):
    d = pdeg_ref[0][:N, 0:1] + pdeg_ref[1][:N, 0:1]
    rid = lax.broadcasted_iota(jnp.int32, (N, 1), 0)
    d = d - jnp.where(rid < PAD, 1.0, 0.0)
    return lax.rsqrt(d + 1.0)


def _k1_body(x_ref, w_ref, pdeg_ref, g_ref):
    dinv = _dinv_from(pdeg_ref)
    h = jnp.dot(x_ref[...], w_ref[...], preferred_element_type=jnp.float32)
    g_ref[pl.ds(0, N)] = h * dinv
    g_ref[pl.ds(N, NT - N)] = jnp.zeros((NT - N, D), jnp.float32)


def _mid_body(p_ref, g_ref, pdeg_ref, w_ref, b_ref, o_ref):
    dinv = _dinv_from(pdeg_ref)
    h = dinv * (p_ref[0][:N] + p_ref[1][:N] + g_ref[:N]) + b_ref[...]
    a = jnp.maximum(h, 0.0)
    o_ref[pl.ds(0, N)] = dinv * jnp.dot(a, w_ref[...], preferred_element_type=jnp.float32)
    o_ref[pl.ds(N, NT - N)] = jnp.zeros((NT - N, D), jnp.float32)


def _readout_body(p_ref, g_ref, pdeg_ref, b3_ref, wr1_ref, br1_ref, wr2_ref,
                  br2_ref, batch_ref, o_ref):
    dinv = _dinv_from(pdeg_ref)
    h = dinv * (p_ref[0][:N] + p_ref[1][:N] + g_ref[:N]) + b3_ref[...]
    t = jnp.maximum(
        jnp.dot(h, wr1_ref[...], preferred_element_type=jnp.float32) + br1_ref[...],
        0.0,
    )
    r = jnp.dot(t, wr2_ref[...], preferred_element_type=jnp.float32) + br2_ref[...]
    onehot = (batch_ref[...] == lax.broadcasted_iota(jnp.int32, (N, NG), 1)
              ).astype(jnp.float32)
    dn = (((0,), (0,)), ((), ()))
    sums = lax.dot_general(onehot, r, dn, preferred_element_type=jnp.float32)
    counts = lax.dot_general(onehot, jnp.ones((N, 1), jnp.float32), dn,
                             preferred_element_type=jnp.float32)
    o_ref[...] = sums / jnp.maximum(counts, 1.0)


def kernel(x, edge_index, batch, W1, b1, W2, b2, W3, b3, Wr1, br1, Wr2, br2):
    src = edge_index[0].astype(jnp.int32)
    dst = edge_index[1].astype(jnp.int32)
    pad = EPAD - E
    srcp = jnp.concatenate([src, jnp.full((pad,), N, jnp.int32)])
    pad_dst = jnp.arange(pad, dtype=jnp.int32) % N
    dstp = jnp.concatenate([dst, pad_dst])
    e0 = NS * K0 * CH
    sd_c0 = jnp.stack([srcp[:e0].reshape(NS, K0, CH),
                       dstp[:e0].reshape(NS, K0, CH)], axis=2)
    sd_c0 = jnp.pad(sd_c0, ((0, 0), (0, K1 - K0), (0, 0), (0, 0)))
    sd_c1 = jnp.stack([srcp[e0:].reshape(NS, K1, CH),
                       dstp[e0:].reshape(NS, K1, CH)], axis=2)
    sdl = jnp.stack([sd_c0, sd_c1])
    zf = jnp.zeros((RPT, D), jnp.float32)

    onest = jnp.concatenate([jnp.ones((N, D), jnp.float32),
                             jnp.zeros((NT - N, D), jnp.float32)])
    pdeg = _edge_degree(onest, sdl, zf)

    g1 = pl.pallas_call(
        _k1_body, out_shape=jax.ShapeDtypeStruct((NT, D), jnp.float32),
    )(x, W1, pdeg)
    p1 = _edge_scatter(g1, sdl, zf)

    mid = pl.pallas_call(
        _mid_body, out_shape=jax.ShapeDtypeStruct((NT, D), jnp.float32),
    )
    g2 = mid(p1, g1, pdeg, W2, b1.reshape(1, D))
    p2 = _edge_scatter(g2, sdl, zf)

    g3 = mid(p2, g2, pdeg, W3, b2.reshape(1, D))
    p3 = _edge_scatter(g3, sdl, zf)

    out = pl.pallas_call(
        _readout_body, out_shape=jax.ShapeDtypeStruct((NG, 1), jnp.float32),
    )(p3, g3, pdeg, b3.reshape(1, D), Wr1, br1.reshape(1, D // 2), Wr2,
      br2.reshape(1, 1), batch.astype(jnp.int32).reshape(N, 1))
    return out


# speedup vs baseline: 1.1105x; 1.1105x over previous
import functools

import jax
import jax.numpy as jnp
from jax import lax
from jax.experimental import pallas as pl
from jax.experimental.pallas import tpu as pltpu
from jax.experimental.pallas import tpu_sc as plsc

N = 10000
E = 320000
D = 128
NG = 64
NC = 2
NS = 16
NW = NC * NS
CH = 128
K = -(-E // (NW * CH))
K0 = 105
K1 = 2 * K - K0
EPAD = NS * CH * (K0 + K1)
RPT = 632
NACC = NS * RPT
NT = N + 8


def _sc_mesh():
    return plsc.VectorSubcoreMesh(
        core_axis_name="c", subcore_axis_name="s", num_cores=NC, num_subcores=NS
    )


def _scatter_body(g_hbm, sdl, zrows, out_hbm, sd0, sd1, rows0, rows1, acc,
                  isem0, isem1, gsem0, gsem1):
    c = lax.axis_index("c")
    s = lax.axis_index("s")
    kc = jnp.where(c == 0, K0, K1)
    pltpu.sync_copy(zrows, acc.at[pl.ds(s * RPT, RPT)])
    plsc.subcore_barrier()

    pltpu.sync_copy(sdl.at[c, s, 0], sd0)
    pltpu.async_copy(g_hbm.at[sd0.at[0]], rows0, gsem0)
    pltpu.async_copy(sdl.at[c, s, 1], sd1, isem1)

    def pair(i, carry):
        k0 = i * 2
        pltpu.make_async_copy(sdl.at[c, s, k0 + 1], sd1, isem1).wait()
        pltpu.async_copy(g_hbm.at[sd1.at[0]], rows1, gsem1)
        pltpu.make_async_copy(g_hbm.at[sd0.at[0]], rows0, gsem0).wait()
        pltpu.sync_copy(rows0, acc.at[sd0.at[1]], add=True)
        pltpu.async_copy(sdl.at[c, s, k0 + 2], sd0, isem0)
        pltpu.make_async_copy(sdl.at[c, s, k0 + 2], sd0, isem0).wait()
        pltpu.async_copy(g_hbm.at[sd0.at[0]], rows0, gsem0)
        pltpu.make_async_copy(g_hbm.at[sd1.at[0]], rows1, gsem1).wait()
        pltpu.sync_copy(rows1, acc.at[sd1.at[1]], add=True)
        pltpu.async_copy(sdl.at[c, s, k0 + 3], sd1, isem1)
        return carry

    lax.fori_loop(0, (kc - 3) // 2, pair, 0)
    pltpu.make_async_copy(sdl.at[c, s, kc - 2], sd1, isem1).wait()
    pltpu.async_copy(g_hbm.at[sd1.at[0]], rows1, gsem1)
    pltpu.make_async_copy(g_hbm.at[sd0.at[0]], rows0, gsem0).wait()
    pltpu.sync_copy(rows0, acc.at[sd0.at[1]], add=True)
    pltpu.sync_copy(sdl.at[c, s, kc - 1], sd0)
    pltpu.async_copy(g_hbm.at[sd0.at[0]], rows0, gsem0)
    pltpu.make_async_copy(g_hbm.at[sd1.at[0]], rows1, gsem1).wait()
    pltpu.sync_copy(rows1, acc.at[sd1.at[1]], add=True)
    pltpu.make_async_copy(g_hbm.at[sd0.at[0]], rows0, gsem0).wait()
    pltpu.sync_copy(rows0, acc.at[sd0.at[1]], add=True)
    plsc.subcore_barrier()
    pltpu.sync_copy(acc.at[pl.ds(s * RPT, RPT)], out_hbm.at[c, pl.ds(s * RPT, RPT)])


def _edge_scatter(g, sdl, zrows):
    return pl.kernel(
        _scatter_body,
        out_type=jax.ShapeDtypeStruct((NC, NACC, D), jnp.float32),
        mesh=_sc_mesh(),
        scratch_types=[
            pltpu.VMEM((2, CH), jnp.int32),
            pltpu.VMEM((2, CH), jnp.int32),
            pltpu.VMEM((CH, D), jnp.float32),
            pltpu.VMEM((CH, D), jnp.float32),
            pltpu.VMEM_SHARED((NACC, D), jnp.float32),
            pltpu.SemaphoreType.DMA,
            pltpu.SemaphoreType.DMA,
            pltpu.SemaphoreType.DMA,
            pltpu.SemaphoreType.DMA,
        ],
    )(g, sdl, zrows)


def _deg_body(ones_hbm, sdl, zrows, out_hbm, sd0, sd1, rows0, acc, isem0, isem1):
    c = lax.axis_index("c")
    s = lax.axis_index("s")
    kc = jnp.where(c == 0, K0, K1)
    pltpu.sync_copy(zrows, acc.at[pl.ds(s * RPT, RPT)])
    pltpu.sync_copy(ones_hbm.at[pl.ds(0, CH)], rows0)
    plsc.subcore_barrier()

    pltpu.sync_copy(sdl.at[c, s, 0], sd0)
    pltpu.async_copy(sdl.at[c, s, 1], sd1, isem1)

    def pair(i, carry):
        k0 = i * 2
        pltpu.sync_copy(rows0, acc.at[sd0.at[1]], add=True)
        pltpu.async_copy(sdl.at[c, s, k0 + 2], sd0, isem0)
        pltpu.make_async_copy(sdl.at[c, s, k0 + 1], sd1, isem1).wait()
        pltpu.sync_copy(rows0, acc.at[sd1.at[1]], add=True)
        pltpu.async_copy(sdl.at[c, s, k0 + 3], sd1, isem1)
        pltpu.make_async_copy(sdl.at[c, s, k0 + 2], sd0, isem0).wait()
        return carry

    lax.fori_loop(0, (kc - 3) // 2, pair, 0)
    pltpu.sync_copy(rows0, acc.at[sd0.at[1]], add=True)
    pltpu.make_async_copy(sdl.at[c, s, kc - 2], sd1, isem1).wait()
    pltpu.sync_copy(rows0, acc.at[sd1.at[1]], add=True)
    pltpu.sync_copy(sdl.at[c, s, kc - 1], sd0)
    pltpu.sync_copy(rows0, acc.at[sd0.at[1]], add=True)
    plsc.subcore_barrier()
    pltpu.sync_copy(acc.at[pl.ds(s * RPT, RPT)], out_hbm.at[c, pl.ds(s * RPT, RPT)])


def _edge_degree(ones, sdl, zrows):
    return pl.kernel(
        _deg_body,
        out_type=jax.ShapeDtypeStruct((NC, NACC, D), jnp.float32),
        mesh=_sc_mesh(),
        scratch_types=[
            pltpu.VMEM((2, CH), jnp.int32),
            pltpu.VMEM((2, CH), jnp.int32),
            pltpu.VMEM((CH, D), jnp.float32),
            pltpu.VMEM_SHARED((NACC, D), jnp.float32),
            pltpu.SemaphoreType.DMA,
            pltpu.SemaphoreType.DMA,
        ],
    )(ones, sdl, zrows)


PAD = EPAD - E


def _dinv_from(pdeg_ref):
    d = pdeg_ref[0][:N, 0:1] + pdeg_ref[1][:N, 0:1]
    rid = lax.broadcasted_iota(jnp.int32, (N, 1), 0)
    d = d - jnp.where(rid < PAD, 1.0, 0.0)
    return lax.rsqrt(d + 1.0)


def _k1_body(x_ref, w_ref, pdeg_ref, g_ref):
    dinv = _dinv_from(pdeg_ref)
    h = jnp.dot(x_ref[...], w_ref[...], preferred_element_type=jnp.float32)
    g_ref[pl.ds(0, N)] = h * dinv
    g_ref[pl.ds(N, NT - N)] = jnp.zeros((NT - N, D), jnp.float32)


def _mid_body(p_ref, g_ref, pdeg_ref, w_ref, b_ref, o_ref):
    dinv = _dinv_from(pdeg_ref)
    h = dinv * (p_ref[0][:N] + p_ref[1][:N] + g_ref[:N]) + b_ref[...]
    a = jnp.maximum(h, 0.0)
    o_ref[pl.ds(0, N)] = dinv * jnp.dot(a, w_ref[...], preferred_element_type=jnp.float32)
    o_ref[pl.ds(N, NT - N)] = jnp.zeros((NT - N, D), jnp.float32)


def _readout_body(p_ref, g_ref, pdeg_ref, b3_ref, wr1_ref, br1_ref, wr2_ref,
                  br2_ref, batch_ref, o_ref):
    dinv = _dinv_from(pdeg_ref)
    h = dinv * (p_ref[0][:N] + p_ref[1][:N] + g_ref[:N]) + b3_ref[...]
    t = jnp.maximum(
        jnp.dot(h, wr1_ref[...], preferred_element_type=jnp.float32) + br1_ref[...],
        0.0,
    )
    r = jnp.dot(t, wr2_ref[...], preferred_element_type=jnp.float32) + br2_ref[...]
    onehot = (batch_ref[...] == lax.broadcasted_iota(jnp.int32, (N, NG), 1)
              ).astype(jnp.float32)
    dn = (((0,), (0,)), ((), ()))
    sums = lax.dot_general(onehot, r, dn, preferred_element_type=jnp.float32)
    counts = lax.dot_general(onehot, jnp.ones((N, 1), jnp.float32), dn,
                             preferred_element_type=jnp.float32)
    o_ref[...] = sums / jnp.maximum(counts, 1.0)


def kernel(x, edge_index, batch, W1, b1, W2, b2, W3, b3, Wr1, br1, Wr2, br2):
    src = edge_index[0].astype(jnp.int32)
    dst = edge_index[1].astype(jnp.int32)
    pad = EPAD - E
    srcp = jnp.concatenate([src, jnp.full((pad,), N, jnp.int32)])
    pad_dst = jnp.arange(pad, dtype=jnp.int32) % N
    dstp = jnp.concatenate([dst, pad_dst])
    e0 = NS * K0 * CH
    kmax = max(K0, K1)
    sd_c0 = jnp.stack([srcp[:e0].reshape(NS, K0, CH),
                       dstp[:e0].reshape(NS, K0, CH)], axis=2)
    sd_c0 = jnp.pad(sd_c0, ((0, 0), (0, kmax - K0), (0, 0), (0, 0)))
    sd_c1 = jnp.stack([srcp[e0:].reshape(NS, K1, CH),
                       dstp[e0:].reshape(NS, K1, CH)], axis=2)
    sd_c1 = jnp.pad(sd_c1, ((0, 0), (0, kmax - K1), (0, 0), (0, 0)))
    sdl = jnp.stack([sd_c0, sd_c1])
    zf = jnp.zeros((RPT, D), jnp.float32)

    onest = jnp.concatenate([jnp.ones((N, D), jnp.float32),
                             jnp.zeros((NT - N, D), jnp.float32)])
    pdeg = _edge_degree(onest, sdl, zf)

    g1 = pl.pallas_call(
        _k1_body, out_shape=jax.ShapeDtypeStruct((NT, D), jnp.float32),
    )(x, W1, pdeg)
    p1 = _edge_scatter(g1, sdl, zf)

    mid = pl.pallas_call(
        _mid_body, out_shape=jax.ShapeDtypeStruct((NT, D), jnp.float32),
    )
    g2 = mid(p1, g1, pdeg, W2, b1.reshape(1, D))
    p2 = _edge_scatter(g2, sdl, zf)

    g3 = mid(p2, g2, pdeg, W3, b2.reshape(1, D))
    p3 = _edge_scatter(g3, sdl, zf)

    out = pl.pallas_call(
        _readout_body, out_shape=jax.ShapeDtypeStruct((NG, 1), jnp.float32),
    )(p3, g3, pdeg, b3.reshape(1, D), Wr1, br1.reshape(1, D // 2), Wr2,
      br2.reshape(1, 1), batch.astype(jnp.int32).reshape(N, 1))
    return out


# speedup vs baseline: 1.3937x; 1.2549x over previous
import functools

import jax
import jax.numpy as jnp
from jax import lax
from jax.experimental import pallas as pl
from jax.experimental.pallas import tpu as pltpu
from jax.experimental.pallas import tpu_sc as plsc

N = 10000
E = 320000
D = 128
NG = 64
NC = 2
NS = 16
NW = NC * NS
CH = 120
K = 84
EPAD = NW * CH * K
RPT = 632
NACC = NS * RPT
NT = N + 8


def _sc_mesh():
    return plsc.VectorSubcoreMesh(
        core_axis_name="c", subcore_axis_name="s", num_cores=NC, num_subcores=NS
    )


def _scatter_body(g_hbm, sdl, zrows, out_hbm, sd0, sd1, sd2, rows0, rows1,
                  rows2, acc, isem0, isem1, isem2, gsem0, gsem1, gsem2):
    c = lax.axis_index("c")
    s = lax.axis_index("s")
    pltpu.sync_copy(zrows, acc.at[pl.ds(s * RPT, RPT)])
    plsc.subcore_barrier()

    pltpu.sync_copy(sdl.at[c, s, 0], sd0)
    pltpu.async_copy(g_hbm.at[sd0.at[0]], rows0, gsem0)
    pltpu.sync_copy(sdl.at[c, s, 1], sd1)
    pltpu.async_copy(g_hbm.at[sd1.at[0]], rows1, gsem1)
    pltpu.async_copy(sdl.at[c, s, 2], sd2, isem2)

    def triple(i, carry):
        k0 = i * 3
        pltpu.make_async_copy(sdl.at[c, s, k0 + 2], sd2, isem2).wait()
        pltpu.async_copy(g_hbm.at[sd2.at[0]], rows2, gsem2)
        pltpu.make_async_copy(g_hbm.at[sd0.at[0]], rows0, gsem0).wait()
        pltpu.sync_copy(rows0, acc.at[sd0.at[1]], add=True)
        pltpu.async_copy(sdl.at[c, s, k0 + 3], sd0, isem0)
        pltpu.make_async_copy(sdl.at[c, s, k0 + 3], sd0, isem0).wait()
        pltpu.async_copy(g_hbm.at[sd0.at[0]], rows0, gsem0)
        pltpu.make_async_copy(g_hbm.at[sd1.at[0]], rows1, gsem1).wait()
        pltpu.sync_copy(rows1, acc.at[sd1.at[1]], add=True)
        pltpu.async_copy(sdl.at[c, s, k0 + 4], sd1, isem1)
        pltpu.make_async_copy(sdl.at[c, s, k0 + 4], sd1, isem1).wait()
        pltpu.async_copy(g_hbm.at[sd1.at[0]], rows1, gsem1)
        pltpu.make_async_copy(g_hbm.at[sd2.at[0]], rows2, gsem2).wait()
        pltpu.sync_copy(rows2, acc.at[sd2.at[1]], add=True)
        pltpu.async_copy(sdl.at[c, s, k0 + 5], sd2, isem2)
        return carry

    lax.fori_loop(0, (K - 3) // 3, triple, 0)
    pltpu.make_async_copy(sdl.at[c, s, K - 1], sd2, isem2).wait()
    pltpu.async_copy(g_hbm.at[sd2.at[0]], rows2, gsem2)
    pltpu.make_async_copy(g_hbm.at[sd0.at[0]], rows0, gsem0).wait()
    pltpu.sync_copy(rows0, acc.at[sd0.at[1]], add=True)
    pltpu.make_async_copy(g_hbm.at[sd1.at[0]], rows1, gsem1).wait()
    pltpu.sync_copy(rows1, acc.at[sd1.at[1]], add=True)
    pltpu.make_async_copy(g_hbm.at[sd2.at[0]], rows2, gsem2).wait()
    pltpu.sync_copy(rows2, acc.at[sd2.at[1]], add=True)
    plsc.subcore_barrier()
    pltpu.sync_copy(acc.at[pl.ds(s * RPT, RPT)], out_hbm.at[c, pl.ds(s * RPT, RPT)])


def _edge_scatter(g, sdl, zrows):
    return pl.kernel(
        _scatter_body,
        out_type=jax.ShapeDtypeStruct((NC, NACC, D), jnp.float32),
        mesh=_sc_mesh(),
        scratch_types=[
            pltpu.VMEM((2, CH), jnp.int32),
            pltpu.VMEM((2, CH), jnp.int32),
            pltpu.VMEM((2, CH), jnp.int32),
            pltpu.VMEM((CH, D), jnp.float32),
            pltpu.VMEM((CH, D), jnp.float32),
            pltpu.VMEM((CH, D), jnp.float32),
            pltpu.VMEM_SHARED((NACC, D), jnp.float32),
            pltpu.SemaphoreType.DMA,
            pltpu.SemaphoreType.DMA,
            pltpu.SemaphoreType.DMA,
            pltpu.SemaphoreType.DMA,
            pltpu.SemaphoreType.DMA,
            pltpu.SemaphoreType.DMA,
        ],
    )(g, sdl, zrows)


def _deg_body(ones_hbm, sdl, zrows, out_hbm, sd0, sd1, rows0, acc, isem0, isem1):
    c = lax.axis_index("c")
    s = lax.axis_index("s")
    pltpu.sync_copy(zrows, acc.at[pl.ds(s * RPT, RPT)])
    pltpu.sync_copy(ones_hbm.at[pl.ds(0, CH)], rows0)
    plsc.subcore_barrier()

    pltpu.sync_copy(sdl.at[c, s, 0], sd0)
    pltpu.async_copy(sdl.at[c, s, 1], sd1, isem1)

    def pair(i, carry):
        k0 = i * 2
        pltpu.sync_copy(rows0, acc.at[sd0.at[1]], add=True)
        pltpu.async_copy(sdl.at[c, s, k0 + 2], sd0, isem0)
        pltpu.make_async_copy(sdl.at[c, s, k0 + 1], sd1, isem1).wait()
        pltpu.sync_copy(rows0, acc.at[sd1.at[1]], add=True)
        pltpu.async_copy(sdl.at[c, s, k0 + 3], sd1, isem1)
        pltpu.make_async_copy(sdl.at[c, s, k0 + 2], sd0, isem0).wait()
        return carry

    lax.fori_loop(0, (K - 4) // 2, pair, 0)
    pltpu.sync_copy(rows0, acc.at[sd0.at[1]], add=True)
    pltpu.async_copy(sdl.at[c, s, K - 2], sd0, isem0)
    pltpu.make_async_copy(sdl.at[c, s, K - 3], sd1, isem1).wait()
    pltpu.sync_copy(rows0, acc.at[sd1.at[1]], add=True)
    pltpu.make_async_copy(sdl.at[c, s, K - 2], sd0, isem0).wait()
    pltpu.sync_copy(rows0, acc.at[sd0.at[1]], add=True)
    pltpu.sync_copy(sdl.at[c, s, K - 1], sd1)
    pltpu.sync_copy(rows0, acc.at[sd1.at[1]], add=True)
    plsc.subcore_barrier()
    pltpu.sync_copy(acc.at[pl.ds(s * RPT, RPT)], out_hbm.at[c, pl.ds(s * RPT, RPT)])


def _edge_degree(ones, sdl, zrows):
    return pl.kernel(
        _deg_body,
        out_type=jax.ShapeDtypeStruct((NC, NACC, D), jnp.float32),
        mesh=_sc_mesh(),
        scratch_types=[
            pltpu.VMEM((2, CH), jnp.int32),
            pltpu.VMEM((2, CH), jnp.int32),
            pltpu.VMEM((CH, D), jnp.float32),
            pltpu.VMEM_SHARED((NACC, D), jnp.float32),
            pltpu.SemaphoreType.DMA,
            pltpu.SemaphoreType.DMA,
        ],
    )(ones, sdl, zrows)


PAD = EPAD - E


def _dinv_from(pdeg_ref):
    d = pdeg_ref[0][:N, 0:1] + pdeg_ref[1][:N, 0:1]
    rid = lax.broadcasted_iota(jnp.int32, (N, 1), 0)
    d = d - jnp.where(rid < PAD, 1.0, 0.0)
    return lax.rsqrt(d + 1.0)


def _k1_body(x_ref, w_ref, pdeg_ref, g_ref):
    dinv = _dinv_from(pdeg_ref)
    h = jnp.dot(x_ref[...], w_ref[...], preferred_element_type=jnp.float32)
    g_ref[pl.ds(0, N)] = h * dinv
    g_ref[pl.ds(N, NT - N)] = jnp.zeros((NT - N, D), jnp.float32)


def _mid_body(p_ref, g_ref, pdeg_ref, w_ref, b_ref, o_ref):
    dinv = _dinv_from(pdeg_ref)
    h = dinv * (p_ref[0][:N] + p_ref[1][:N] + g_ref[:N]) + b_ref[...]
    a = jnp.maximum(h, 0.0)
    o_ref[pl.ds(0, N)] = dinv * jnp.dot(a, w_ref[...], preferred_element_type=jnp.float32)
    o_ref[pl.ds(N, NT - N)] = jnp.zeros((NT - N, D), jnp.float32)


def _readout_body(p_ref, g_ref, pdeg_ref, b3_ref, wr1_ref, br1_ref, wr2_ref,
                  br2_ref, batch_ref, o_ref):
    dinv = _dinv_from(pdeg_ref)
    h = dinv * (p_ref[0][:N] + p_ref[1][:N] + g_ref[:N]) + b3_ref[...]
    t = jnp.maximum(
        jnp.dot(h, wr1_ref[...], preferred_element_type=jnp.float32) + br1_ref[...],
        0.0,
    )
    r = jnp.dot(t, wr2_ref[...], preferred_element_type=jnp.float32) + br2_ref[...]
    onehot = (batch_ref[...] == lax.broadcasted_iota(jnp.int32, (N, NG), 1)
              ).astype(jnp.float32)
    dn = (((0,), (0,)), ((), ()))
    sums = lax.dot_general(onehot, r, dn, preferred_element_type=jnp.float32)
    counts = lax.dot_general(onehot, jnp.ones((N, 1), jnp.float32), dn,
                             preferred_element_type=jnp.float32)
    o_ref[...] = sums / jnp.maximum(counts, 1.0)


def kernel(x, edge_index, batch, W1, b1, W2, b2, W3, b3, Wr1, br1, Wr2, br2):
    src = edge_index[0].astype(jnp.int32)
    dst = edge_index[1].astype(jnp.int32)
    pad = EPAD - E
    srcp = jnp.concatenate([src, jnp.full((pad,), N, jnp.int32)])
    pad_dst = jnp.arange(pad, dtype=jnp.int32) % N
    dstp = jnp.concatenate([dst, pad_dst])
    sdl = jnp.stack([srcp.reshape(NC, NS, K, CH),
                     dstp.reshape(NC, NS, K, CH)], axis=3)
    zf = jnp.zeros((RPT, D), jnp.float32)

    onest = jnp.concatenate([jnp.ones((N, D), jnp.float32),
                             jnp.zeros((NT - N, D), jnp.float32)])
    pdeg = _edge_degree(onest, sdl, zf)

    g1 = pl.pallas_call(
        _k1_body, out_shape=jax.ShapeDtypeStruct((NT, D), jnp.float32),
    )(x, W1, pdeg)
    p1 = _edge_scatter(g1, sdl, zf)

    mid = pl.pallas_call(
        _mid_body, out_shape=jax.ShapeDtypeStruct((NT, D), jnp.float32),
    )
    g2 = mid(p1, g1, pdeg, W2, b1.reshape(1, D))
    p2 = _edge_scatter(g2, sdl, zf)

    g3 = mid(p2, g2, pdeg, W3, b2.reshape(1, D))
    p3 = _edge_scatter(g3, sdl, zf)

    out = pl.pallas_call(
        _readout_body, out_shape=jax.ShapeDtypeStruct((NG, 1), jnp.float32),
    )(p3, g3, pdeg, b3.reshape(1, D), Wr1, br1.reshape(1, D // 2), Wr2,
      br2.reshape(1, 1), batch.astype(jnp.int32).reshape(N, 1))
    return out
